# baseline (speedup 1.0000x reference)
import functools

import jax
import jax.numpy as jnp
from jax import lax
from jax.experimental import pallas as pl
from jax.experimental.pallas import tpu as pltpu
from jax.experimental.pallas import tpu_sc as plsc

N = 10000
NPAD = 10240
E = 160000
EPAD = 163840
B = 16
T = 8
NW = 32
EPW = EPAD // NW
LANES = 16

_SC_PARAMS = pltpu.CompilerParams(needs_layout_passes=False)


@functools.lru_cache(maxsize=None)
def _mesh():
    return plsc.VectorSubcoreMesh(core_axis_name="c", subcore_axis_name="s")


def _worker_id():
    return lax.axis_index("c") * 16 + lax.axis_index("s")


@functools.lru_cache(maxsize=None)
def _make_deg():
    def body(dst_hbm, out_hbm, part_v, dst_v):
        w = _worker_id()

        @plsc.parallel_loop(0, NPAD // LANES, unroll=8)
        def zero_body(i):
            part_v[pl.ds(i * LANES, LANES)] = jnp.zeros((LANES,), jnp.float32)

        pltpu.sync_copy(dst_hbm.at[pl.ds(w * EPW, EPW)], dst_v)
        ones = jnp.ones((LANES,), jnp.float32)

        @plsc.parallel_loop(0, EPW // LANES, unroll=8)
        def edge_body(g):
            d = dst_v[pl.ds(g * LANES, LANES)]
            plsc.addupdate_scatter(part_v, [d], ones)

        pltpu.sync_copy(part_v, out_hbm.at[w])

    return pl.kernel(
        body,
        out_type=jax.ShapeDtypeStruct((NW, NPAD), jnp.float32),
        mesh=_mesh(),
        scratch_types=[
            pltpu.VMEM((NPAD,), jnp.float32),
            pltpu.VMEM((EPW,), jnp.int32),
        ],
        compiler_params=_SC_PARAMS,
    )


@functools.lru_cache(maxsize=None)
def _make_norm():
    def body(src_hbm, dst_hbm, dinv_hbm, out_hbm, dinv_v, src_v, dst_v, norm_v):
        w = _worker_id()
        pltpu.sync_copy(dinv_hbm, dinv_v)
        pltpu.sync_copy(src_hbm.at[pl.ds(w * EPW, EPW)], src_v)
        pltpu.sync_copy(dst_hbm.at[pl.ds(w * EPW, EPW)], dst_v)

        @plsc.parallel_loop(0, EPW // LANES, unroll=4)
        def nb(g):
            idx = pl.ds(g * LANES, LANES)
            s = src_v[idx]
            d = dst_v[idx]
            norm_v[idx] = (plsc.load_gather(dinv_v, [s])
                           * plsc.load_gather(dinv_v, [d]))

        pltpu.sync_copy(norm_v, out_hbm.at[pl.ds(w * EPW, EPW)])

    return pl.kernel(
        body,
        out_type=jax.ShapeDtypeStruct((EPAD,), jnp.float32),
        mesh=_mesh(),
        scratch_types=[
            pltpu.VMEM((NPAD,), jnp.float32),
            pltpu.VMEM((EPW,), jnp.int32),
            pltpu.VMEM((EPW,), jnp.int32),
            pltpu.VMEM((EPW,), jnp.float32),
        ],
        compiler_params=_SC_PARAMS,
    )


def _dinv_body(part_ref, out_ref):
    deg = jnp.sum(part_ref[...], axis=0, keepdims=True) + 1.0
    out_ref[...] = lax.rsqrt(deg)


def _dinv_tc(part):
    bn = 2048
    return pl.pallas_call(
        _dinv_body,
        grid=(NPAD // bn,),
        in_specs=[pl.BlockSpec((NW, bn), lambda j: (0, j))],
        out_specs=pl.BlockSpec((1, bn), lambda j: (0, j)),
        out_shape=jax.ShapeDtypeStruct((1, NPAD), jnp.float32),
    )(part)


def _mm_body(a_ref, b_ref, o_ref):
    o_ref[...] = jnp.dot(a_ref[...], b_ref[...],
                         preferred_element_type=jnp.float32)


def _mm(wt, xh):
    ho, hi = wt.shape
    bn = 2048
    return pl.pallas_call(
        _mm_body,
        grid=(NPAD // bn,),
        in_specs=[
            pl.BlockSpec((ho, hi), lambda j: (0, 0)),
            pl.BlockSpec((hi, bn), lambda j: (0, j)),
        ],
        out_specs=pl.BlockSpec((ho, bn), lambda j: (0, j)),
        out_shape=jax.ShapeDtypeStruct((ho, NPAD), jnp.float32),
    )(wt, xh)


_CH = 8192


@functools.lru_cache(maxsize=None)
def _make_prop(h_out, r_res, pool):
    sweeps = h_out // (NW * r_res)
    assert sweeps * r_res * NW == h_out

    scratch = (
        [pltpu.VMEM((NPAD,), jnp.float32) for _ in range(2 * r_res)]
        + [
            pltpu.VMEM((NPAD,), jnp.float32),
            pltpu.VMEM((h_out,), jnp.float32),
            pltpu.VMEM((_CH,), jnp.int32),
            pltpu.VMEM((_CH,), jnp.int32),
            pltpu.VMEM((_CH,), jnp.float32),
        ]
    )
    if pool:
        scratch += [
            pltpu.VMEM((NPAD,), jnp.int32),
            pltpu.VMEM((32,), jnp.float32),
        ]
        out_type = jax.ShapeDtypeStruct((h_out, 32), jnp.float32)
    else:
        out_type = jax.ShapeDtypeStruct((h_out, NPAD), jnp.float32)

    def body(*refs):
        if pool:
            (z_hbm, dinv_hbm, norm_hbm, b_hbm, src_hbm, dst_hbm, batch_hbm,
             out_hbm) = refs[:8]
            sc = refs[8:]
        else:
            (z_hbm, dinv_hbm, norm_hbm, b_hbm, src_hbm, dst_hbm,
             out_hbm) = refs[:7]
            sc = refs[7:]
        ins = sc[:r_res]
        outs = sc[r_res:2 * r_res]
        dinv_v = sc[2 * r_res]
        b_v = sc[2 * r_res + 1]
        src_v = sc[2 * r_res + 2]
        dst_v = sc[2 * r_res + 3]
        norm_v = sc[2 * r_res + 4]
        if pool:
            batch_v = sc[2 * r_res + 5]
            acc_v = sc[2 * r_res + 6]

        w = _worker_id()
        pltpu.sync_copy(dinv_hbm, dinv_v)
        pltpu.sync_copy(b_hbm, b_v)
        if pool:
            pltpu.sync_copy(batch_hbm, batch_v)

        for s in range(sweeps):
            feats = [(s * r_res + r) * NW + w for r in range(r_res)]
            for r in range(r_res):
                pltpu.sync_copy(z_hbm.at[feats[r]], ins[r])

            @plsc.parallel_loop(0, NPAD // LANES, unroll=4)
            def init_body(i):
                idx = pl.ds(i * LANES, LANES)
                dv = dinv_v[idx]
                d2 = dv * dv
                for r in range(r_res):
                    outs[r][idx] = ins[r][idx] * d2

            def chunk_body(c, _):
                pltpu.sync_copy(src_hbm.at[pl.ds(c * _CH, _CH)], src_v)
                pltpu.sync_copy(dst_hbm.at[pl.ds(c * _CH, _CH)], dst_v)
                pltpu.sync_copy(norm_hbm.at[pl.ds(c * _CH, _CH)], norm_v)

                @plsc.parallel_loop(0, _CH // LANES, unroll=8)
                def edge_body(g):
                    idx = pl.ds(g * LANES, LANES)
                    sidx = src_v[idx]
                    didx = dst_v[idx]
                    nm = norm_v[idx]
                    for r in range(r_res):
                        msg = plsc.load_gather(ins[r], [sidx]) * nm
                        plsc.addupdate_scatter(outs[r], [didx], msg)

                return 0

            lax.fori_loop(0, EPAD // _CH, chunk_body, 0)

            for r in range(r_res):
                fsplat = jnp.zeros((LANES,), jnp.int32) + feats[r]
                bvec = plsc.load_gather(b_v, [fsplat])
                if pool:
                    acc_v[pl.ds(0, LANES)] = jnp.zeros((LANES,), jnp.float32)
                    acc_v[pl.ds(LANES, LANES)] = jnp.zeros((LANES,), jnp.float32)

                    @plsc.parallel_loop(0, NPAD // LANES, unroll=4)
                    def ep_body(i):
                        idx = pl.ds(i * LANES, LANES)
                        vals = jnp.maximum(outs[r][idx] + bvec, 0.0)
                        ib = batch_v[idx]
                        plsc.addupdate_scatter(acc_v, [ib], vals)

                    pltpu.sync_copy(acc_v, out_hbm.at[feats[r]])
                else:
                    @plsc.parallel_loop(0, NPAD // LANES, unroll=4)
                    def ep_body(i):
                        idx = pl.ds(i * LANES, LANES)
                        outs[r][idx] = jnp.maximum(outs[r][idx] + bvec, 0.0)

                    pltpu.sync_copy(outs[r], out_hbm.at[feats[r]])

    return pl.kernel(
        body,
        out_type=out_type,
        mesh=_mesh(),
        scratch_types=scratch,
        compiler_params=_SC_PARAMS,
    )


def _seq_body(seq_ref, batch_ref, wih_ref, whh_ref, bih_ref, bhh_ref,
              fcw_ref, fcb_ref, out_ref):
    batch3 = batch_ref[...][None, :, :]
    seg = lax.broadcasted_iota(jnp.int32, (B, NPAD // 128, 128), 0)
    cnt = jnp.sum(jnp.where(batch3 == seg, 1.0, 0.0), axis=(1, 2))
    cntc = jnp.maximum(cnt, 1.0).reshape(B, 1)

    bias = (bih_ref[...] + bhh_ref[...])[None, :]
    wih = wih_ref[...]
    whh = whh_ref[...]
    fcw = fcw_ref[...]
    fcb = fcb_ref[...][None, :]

    h = jnp.zeros((B, 128), jnp.float32)
    c = jnp.zeros((B, 128), jnp.float32)
    dn = (((1,), (1,)), ((), ()))
    for t in range(T):
        xt = seq_ref[t] / cntc
        g = (lax.dot_general(xt, wih, dn, preferred_element_type=jnp.float32)
             + lax.dot_general(h, whh, dn, preferred_element_type=jnp.float32)
             + bias)
        i_ = jax.nn.sigmoid(g[:, 0:128])
        f_ = jax.nn.sigmoid(g[:, 128:256])
        g_ = jnp.tanh(g[:, 256:384])
        o_ = jax.nn.sigmoid(g[:, 384:512])
        c = f_ * c + i_ * g_
        h = o_ * jnp.tanh(c)
        out_ref[t] = jnp.dot(h, fcw, preferred_element_type=jnp.float32) + fcb


def _seq_tc(seq, batch80, w_ih, w_hh, b_ih, b_hh, fc_w, fc_b):
    return pl.pallas_call(
        _seq_body,
        out_shape=jax.ShapeDtypeStruct((T, B, 64), jnp.float32),
    )(seq, batch80, w_ih, w_hh, b_ih, b_hh, fc_w, fc_b)


def kernel(x, W1, b1, W2, b2, W3, b3, W4, b4, W_ih, W_hh, b_ih, b_hh,
           fc_W, fc_b, edge_index, batch):
    xp = jnp.pad(x, ((0, 0), (0, NPAD - N), (0, 0)))
    xt = jnp.swapaxes(xp, 1, 2)
    src = jnp.pad(edge_index[:, 0, :], ((0, 0), (0, EPAD - E)),
                  constant_values=NPAD - 1)
    dst = jnp.pad(edge_index[:, 1, :], ((0, 0), (0, EPAD - E)),
                  constant_values=NPAD - 1)
    batch_pad = jnp.pad(batch, (0, NPAD - N), constant_values=B)
    batch80 = batch_pad.reshape(NPAD // 128, 128)

    w1t, w2t, w3t, w4t = W1.T, W2.T, W3.T, W4.T

    deg_k = _make_deg()
    norm_k = _make_norm()
    prop1 = _make_prop(512, 4, False)
    prop2 = _make_prop(256, 4, False)
    prop3 = _make_prop(128, 4, False)
    prop4 = _make_prop(64, 2, True)

    def step(_, ts):
        xt_t, src_t, dst_t = ts
        part = deg_k(dst_t)
        dinv = _dinv_tc(part).reshape(NPAD)
        norm = norm_k(src_t, dst_t, dinv)
        z1 = _mm(w1t, xt_t)
        h1 = prop1(z1, dinv, norm, b1, src_t, dst_t)
        z2 = _mm(w2t, h1)
        h2 = prop2(z2, dinv, norm, b2, src_t, dst_t)
        z3 = _mm(w3t, h2)
        h3 = prop3(z3, dinv, norm, b3, src_t, dst_t)
        z4 = _mm(w4t, h3)
        pool = prop4(z4, dinv, norm, b4, src_t, dst_t, batch_pad)
        return 0, pool

    _, pools = lax.scan(step, 0, (xt, src, dst))
    seq_sums = jnp.transpose(pools, (0, 2, 1))[:, :B, :]

    out8 = _seq_tc(seq_sums, batch80, W_ih, W_hh, b_ih, b_hh, fc_W, fc_b)
    return jnp.swapaxes(out8, 0, 1)


# speedup vs baseline: 1.4187x; 1.4187x over previous
import functools

import jax
import jax.numpy as jnp
from jax import lax
from jax.experimental import pallas as pl
from jax.experimental.pallas import tpu as pltpu
from jax.experimental.pallas import tpu_sc as plsc

N = 10000
NPAD = 10240
E = 160000
EPAD = 163840
B = 16
T = 8
NW = 32
EPW = EPAD // NW
LANES = 16

_SC_PARAMS = pltpu.CompilerParams(needs_layout_passes=False)


@functools.lru_cache(maxsize=None)
def _mesh():
    return plsc.VectorSubcoreMesh(core_axis_name="c", subcore_axis_name="s")


def _worker_id():
    return lax.axis_index("c") * 16 + lax.axis_index("s")


@functools.lru_cache(maxsize=None)
def _make_deg():
    def body(dst_hbm, out_hbm, part_v, dst_v):
        w = _worker_id()

        @plsc.parallel_loop(0, NPAD // LANES, unroll=8)
        def zero_body(i):
            part_v[pl.ds(i * LANES, LANES)] = jnp.zeros((LANES,), jnp.float32)

        pltpu.sync_copy(dst_hbm.at[pl.ds(w * EPW, EPW)], dst_v)
        ones = jnp.ones((LANES,), jnp.float32)

        @plsc.parallel_loop(0, EPW // LANES, unroll=8)
        def edge_body(g):
            d = dst_v[pl.ds(g * LANES, LANES)]
            plsc.addupdate_scatter(part_v, [d], ones)

        pltpu.sync_copy(part_v, out_hbm.at[w])

    return pl.kernel(
        body,
        out_type=jax.ShapeDtypeStruct((NW, NPAD), jnp.float32),
        mesh=_mesh(),
        scratch_types=[
            pltpu.VMEM((NPAD,), jnp.float32),
            pltpu.VMEM((EPW,), jnp.int32),
        ],
        compiler_params=_SC_PARAMS,
    )


@functools.lru_cache(maxsize=None)
def _make_norm():
    def body(src_hbm, dst_hbm, dinv_hbm, out_hbm, dinv_v, src_v, dst_v, norm_v):
        w = _worker_id()
        pltpu.sync_copy(dinv_hbm, dinv_v)
        pltpu.sync_copy(src_hbm.at[pl.ds(w * EPW, EPW)], src_v)
        pltpu.sync_copy(dst_hbm.at[pl.ds(w * EPW, EPW)], dst_v)

        @plsc.parallel_loop(0, EPW // LANES, unroll=4)
        def nb(g):
            idx = pl.ds(g * LANES, LANES)
            s = src_v[idx]
            d = dst_v[idx]
            norm_v[idx] = (plsc.load_gather(dinv_v, [s])
                           * plsc.load_gather(dinv_v, [d]))

        pltpu.sync_copy(norm_v, out_hbm.at[pl.ds(w * EPW, EPW)])

    return pl.kernel(
        body,
        out_type=jax.ShapeDtypeStruct((EPAD,), jnp.float32),
        mesh=_mesh(),
        scratch_types=[
            pltpu.VMEM((NPAD,), jnp.float32),
            pltpu.VMEM((EPW,), jnp.int32),
            pltpu.VMEM((EPW,), jnp.int32),
            pltpu.VMEM((EPW,), jnp.float32),
        ],
        compiler_params=_SC_PARAMS,
    )


def _dinv_body(part_ref, out_ref):
    deg = jnp.sum(part_ref[...], axis=0, keepdims=True) + 1.0
    out_ref[...] = lax.rsqrt(deg)


def _dinv_tc(part):
    bn = 2048
    return pl.pallas_call(
        _dinv_body,
        grid=(NPAD // bn,),
        in_specs=[pl.BlockSpec((NW, bn), lambda j: (0, j))],
        out_specs=pl.BlockSpec((1, bn), lambda j: (0, j)),
        out_shape=jax.ShapeDtypeStruct((1, NPAD), jnp.float32),
    )(part)


def _mm_body(a_ref, b_ref, o_ref):
    o_ref[...] = jnp.dot(a_ref[...], b_ref[...],
                         preferred_element_type=jnp.float32)


def _mm(wt, xh):
    ho, hi = wt.shape
    bn = 2048
    return pl.pallas_call(
        _mm_body,
        grid=(NPAD // bn,),
        in_specs=[
            pl.BlockSpec((ho, hi), lambda j: (0, 0)),
            pl.BlockSpec((hi, bn), lambda j: (0, j)),
        ],
        out_specs=pl.BlockSpec((ho, bn), lambda j: (0, j)),
        out_shape=jax.ShapeDtypeStruct((ho, NPAD), jnp.float32),
    )(wt, xh)


_CH = 4096


@functools.lru_cache(maxsize=None)
def _make_prop(h_out, r_res, pool):
    sweeps = h_out // (NW * r_res)
    assert sweeps * r_res * NW == h_out

    scratch = (
        [pltpu.VMEM((NPAD,), jnp.float32) for _ in range(2 * r_res)]
        + [
            pltpu.VMEM((NPAD,), jnp.float32),
            pltpu.VMEM((h_out,), jnp.float32),
            pltpu.VMEM((_CH,), jnp.int32),
            pltpu.VMEM((_CH,), jnp.int32),
            pltpu.VMEM((_CH,), jnp.float32),
            pltpu.VMEM((_CH,), jnp.int32),
            pltpu.VMEM((_CH,), jnp.int32),
            pltpu.VMEM((_CH,), jnp.float32),
            pltpu.SemaphoreType.DMA,
            pltpu.SemaphoreType.DMA,
        ]
    )
    if pool:
        scratch += [
            pltpu.VMEM((NPAD,), jnp.int32),
            pltpu.VMEM((32,), jnp.float32),
        ]
        out_type = jax.ShapeDtypeStruct((h_out, 32), jnp.float32)
    else:
        out_type = jax.ShapeDtypeStruct((h_out, NPAD), jnp.float32)

    def body(*refs):
        if pool:
            (z_hbm, dinv_hbm, norm_hbm, b_hbm, src_hbm, dst_hbm, batch_hbm,
             out_hbm) = refs[:8]
            sc = refs[8:]
        else:
            (z_hbm, dinv_hbm, norm_hbm, b_hbm, src_hbm, dst_hbm,
             out_hbm) = refs[:7]
            sc = refs[7:]
        ins = sc[:r_res]
        outs = sc[r_res:2 * r_res]
        dinv_v = sc[2 * r_res]
        b_v = sc[2 * r_res + 1]
        ebufs = [sc[2 * r_res + 2:2 * r_res + 5],
                 sc[2 * r_res + 5:2 * r_res + 8]]
        sems = [sc[2 * r_res + 8], sc[2 * r_res + 9]]
        if pool:
            batch_v = sc[2 * r_res + 10]
            acc_v = sc[2 * r_res + 11]

        w = _worker_id()
        pltpu.sync_copy(dinv_hbm, dinv_v)
        pltpu.sync_copy(b_hbm, b_v)
        if pool:
            pltpu.sync_copy(batch_hbm, batch_v)

        for s in range(sweeps):
            feats = [(s * r_res + r) * NW + w for r in range(r_res)]
            for r in range(r_res):
                pltpu.sync_copy(z_hbm.at[feats[r]], ins[r])

            @plsc.parallel_loop(0, NPAD // LANES, unroll=4)
            def init_body(i):
                idx = pl.ds(i * LANES, LANES)
                dv = dinv_v[idx]
                d2 = dv * dv
                for r in range(r_res):
                    outs[r][idx] = ins[r][idx] * d2

            nc = EPAD // _CH

            def issue(c, bset):
                pltpu.async_copy(src_hbm.at[pl.ds(c * _CH, _CH)],
                                 ebufs[bset][0], sems[bset])
                pltpu.async_copy(dst_hbm.at[pl.ds(c * _CH, _CH)],
                                 ebufs[bset][1], sems[bset])
                pltpu.async_copy(norm_hbm.at[pl.ds(c * _CH, _CH)],
                                 ebufs[bset][2], sems[bset])

            def drain(bset):
                pltpu.make_async_copy(src_hbm.at[pl.ds(0, _CH)],
                                      ebufs[bset][0], sems[bset]).wait()
                pltpu.make_async_copy(dst_hbm.at[pl.ds(0, _CH)],
                                      ebufs[bset][1], sems[bset]).wait()
                pltpu.make_async_copy(norm_hbm.at[pl.ds(0, _CH)],
                                      ebufs[bset][2], sems[bset]).wait()

            def compute(bset):
                sv, dv, nv = ebufs[bset]

                @plsc.parallel_loop(0, _CH // LANES, unroll=4)
                def edge_body(g):
                    idx = pl.ds(g * LANES, LANES)
                    sidx = sv[idx]
                    didx = dv[idx]
                    nm = nv[idx]
                    for r in range(r_res):
                        msg = plsc.load_gather(ins[r], [sidx]) * nm
                        plsc.addupdate_scatter(outs[r], [didx], msg)

            issue(0, 0)

            def chunk_body(j, _):
                c0 = j * 2
                issue(c0 + 1, 1)
                drain(0)
                compute(0)
                issue(jnp.minimum(c0 + 2, nc - 1), 0)
                drain(1)
                compute(1)
                return 0

            lax.fori_loop(0, nc // 2, chunk_body, 0)
            drain(0)

            for r in range(r_res):
                fsplat = jnp.zeros((LANES,), jnp.int32) + feats[r]
                bvec = plsc.load_gather(b_v, [fsplat])
                if pool:
                    acc_v[pl.ds(0, LANES)] = jnp.zeros((LANES,), jnp.float32)
                    acc_v[pl.ds(LANES, LANES)] = jnp.zeros((LANES,), jnp.float32)

                    @plsc.parallel_loop(0, NPAD // LANES, unroll=4)
                    def ep_body(i):
                        idx = pl.ds(i * LANES, LANES)
                        vals = jnp.maximum(outs[r][idx] + bvec, 0.0)
                        ib = batch_v[idx]
                        plsc.addupdate_scatter(acc_v, [ib], vals)

                    pltpu.sync_copy(acc_v, out_hbm.at[feats[r]])
                else:
                    @plsc.parallel_loop(0, NPAD // LANES, unroll=4)
                    def ep_body(i):
                        idx = pl.ds(i * LANES, LANES)
                        outs[r][idx] = jnp.maximum(outs[r][idx] + bvec, 0.0)

                    pltpu.sync_copy(outs[r], out_hbm.at[feats[r]])

    return pl.kernel(
        body,
        out_type=out_type,
        mesh=_mesh(),
        scratch_types=scratch,
        compiler_params=_SC_PARAMS,
    )


def _seq_body(seq_ref, batch_ref, wih_ref, whh_ref, bih_ref, bhh_ref,
              fcw_ref, fcb_ref, out_ref):
    batch3 = batch_ref[...][None, :, :]
    seg = lax.broadcasted_iota(jnp.int32, (B, NPAD // 128, 128), 0)
    cnt = jnp.sum(jnp.where(batch3 == seg, 1.0, 0.0), axis=(1, 2))
    cntc = jnp.maximum(cnt, 1.0).reshape(B, 1)

    bias = (bih_ref[...] + bhh_ref[...])[None, :]
    wih = wih_ref[...]
    whh = whh_ref[...]
    fcw = fcw_ref[...]
    fcb = fcb_ref[...][None, :]

    h = jnp.zeros((B, 128), jnp.float32)
    c = jnp.zeros((B, 128), jnp.float32)
    dn = (((1,), (1,)), ((), ()))
    for t in range(T):
        xt = seq_ref[t] / cntc
        g = (lax.dot_general(xt, wih, dn, preferred_element_type=jnp.float32)
             + lax.dot_general(h, whh, dn, preferred_element_type=jnp.float32)
             + bias)
        i_ = jax.nn.sigmoid(g[:, 0:128])
        f_ = jax.nn.sigmoid(g[:, 128:256])
        g_ = jnp.tanh(g[:, 256:384])
        o_ = jax.nn.sigmoid(g[:, 384:512])
        c = f_ * c + i_ * g_
        h = o_ * jnp.tanh(c)
        out_ref[t] = jnp.dot(h, fcw, preferred_element_type=jnp.float32) + fcb


def _seq_tc(seq, batch80, w_ih, w_hh, b_ih, b_hh, fc_w, fc_b):
    return pl.pallas_call(
        _seq_body,
        out_shape=jax.ShapeDtypeStruct((T, B, 64), jnp.float32),
    )(seq, batch80, w_ih, w_hh, b_ih, b_hh, fc_w, fc_b)


def kernel(x, W1, b1, W2, b2, W3, b3, W4, b4, W_ih, W_hh, b_ih, b_hh,
           fc_W, fc_b, edge_index, batch):
    xp = jnp.pad(x, ((0, 0), (0, NPAD - N), (0, 0)))
    xt = jnp.swapaxes(xp, 1, 2)
    src = jnp.pad(edge_index[:, 0, :], ((0, 0), (0, EPAD - E)),
                  constant_values=NPAD - 1)
    dst = jnp.pad(edge_index[:, 1, :], ((0, 0), (0, EPAD - E)),
                  constant_values=NPAD - 1)
    batch_pad = jnp.pad(batch, (0, NPAD - N), constant_values=B)
    batch80 = batch_pad.reshape(NPAD // 128, 128)

    w1t, w2t, w3t, w4t = W1.T, W2.T, W3.T, W4.T

    deg_k = _make_deg()
    norm_k = _make_norm()
    prop1 = _make_prop(512, 4, False)
    prop2 = _make_prop(256, 4, False)
    prop3 = _make_prop(128, 4, False)
    prop4 = _make_prop(64, 2, True)

    def step(_, ts):
        xt_t, src_t, dst_t = ts
        part = deg_k(dst_t)
        dinv = _dinv_tc(part).reshape(NPAD)
        norm = norm_k(src_t, dst_t, dinv)
        z1 = _mm(w1t, xt_t)
        h1 = prop1(z1, dinv, norm, b1, src_t, dst_t)
        z2 = _mm(w2t, h1)
        h2 = prop2(z2, dinv, norm, b2, src_t, dst_t)
        z3 = _mm(w3t, h2)
        h3 = prop3(z3, dinv, norm, b3, src_t, dst_t)
        z4 = _mm(w4t, h3)
        pool = prop4(z4, dinv, norm, b4, src_t, dst_t, batch_pad)
        return 0, pool

    _, pools = lax.scan(step, 0, (xt, src, dst))
    seq_sums = jnp.transpose(pools, (0, 2, 1))[:, :B, :]

    out8 = _seq_tc(seq_sums, batch80, W_ih, W_hh, b_ih, b_hh, fc_W, fc_b)
    return jnp.swapaxes(out8, 0, 1)


# speedup vs baseline: 1.4702x; 1.0363x over previous
import functools

import jax
import jax.numpy as jnp
from jax import lax
from jax.experimental import pallas as pl
from jax.experimental.pallas import tpu as pltpu
from jax.experimental.pallas import tpu_sc as plsc

N = 10000
NPAD = 10240
E = 160000
EPAD = 163840
B = 16
T = 8
NW = 32
EPW = EPAD // NW
LANES = 16

_SC_PARAMS = pltpu.CompilerParams(needs_layout_passes=False)


@functools.lru_cache(maxsize=None)
def _mesh():
    return plsc.VectorSubcoreMesh(core_axis_name="c", subcore_axis_name="s")


def _worker_id():
    return lax.axis_index("c") * 16 + lax.axis_index("s")


@functools.lru_cache(maxsize=None)
def _make_deg():
    def body(dst_hbm, out_hbm, part_v, dst_v):
        w = _worker_id()

        @plsc.parallel_loop(0, NPAD // LANES, unroll=8)
        def zero_body(i):
            part_v[pl.ds(i * LANES, LANES)] = jnp.zeros((LANES,), jnp.float32)

        pltpu.sync_copy(dst_hbm.at[pl.ds(w * EPW, EPW)], dst_v)
        ones = jnp.ones((LANES,), jnp.float32)

        @plsc.parallel_loop(0, EPW // LANES, unroll=8)
        def edge_body(g):
            d = dst_v[pl.ds(g * LANES, LANES)]
            plsc.addupdate_scatter(part_v, [d], ones)

        pltpu.sync_copy(part_v, out_hbm.at[w])

    return pl.kernel(
        body,
        out_type=jax.ShapeDtypeStruct((NW, NPAD), jnp.float32),
        mesh=_mesh(),
        scratch_types=[
            pltpu.VMEM((NPAD,), jnp.float32),
            pltpu.VMEM((EPW,), jnp.int32),
        ],
        compiler_params=_SC_PARAMS,
    )


@functools.lru_cache(maxsize=None)
def _make_norm():
    def body(src_hbm, dst_hbm, dinv_hbm, norm_hbm, packed_hbm,
             dinv_v, src_v, dst_v, norm_v, packed_v):
        w = _worker_id()
        pltpu.sync_copy(dinv_hbm, dinv_v)
        pltpu.sync_copy(src_hbm.at[pl.ds(w * EPW, EPW)], src_v)
        pltpu.sync_copy(dst_hbm.at[pl.ds(w * EPW, EPW)], dst_v)

        @plsc.parallel_loop(0, EPW // LANES, unroll=4)
        def nb(g):
            idx = pl.ds(g * LANES, LANES)
            s = src_v[idx]
            d = dst_v[idx]
            norm_v[idx] = (plsc.load_gather(dinv_v, [s])
                           * plsc.load_gather(dinv_v, [d]))
            packed_v[idx] = s | (d << 14)

        pltpu.sync_copy(norm_v, norm_hbm.at[pl.ds(w * EPW, EPW)])
        pltpu.sync_copy(packed_v, packed_hbm.at[pl.ds(w * EPW, EPW)])

    return pl.kernel(
        body,
        out_type=(jax.ShapeDtypeStruct((EPAD,), jnp.float32),
                  jax.ShapeDtypeStruct((EPAD,), jnp.int32)),
        mesh=_mesh(),
        scratch_types=[
            pltpu.VMEM((NPAD,), jnp.float32),
            pltpu.VMEM((EPW,), jnp.int32),
            pltpu.VMEM((EPW,), jnp.int32),
            pltpu.VMEM((EPW,), jnp.float32),
            pltpu.VMEM((EPW,), jnp.int32),
        ],
        compiler_params=_SC_PARAMS,
    )


def _dinv_body(part_ref, out_ref):
    deg = jnp.sum(part_ref[...], axis=0, keepdims=True) + 1.0
    out_ref[...] = lax.rsqrt(deg)


def _dinv_tc(part):
    bn = 2048
    return pl.pallas_call(
        _dinv_body,
        grid=(NPAD // bn,),
        in_specs=[pl.BlockSpec((NW, bn), lambda j: (0, j))],
        out_specs=pl.BlockSpec((1, bn), lambda j: (0, j)),
        out_shape=jax.ShapeDtypeStruct((1, NPAD), jnp.float32),
    )(part)


def _mm_body(a_ref, b_ref, o_ref):
    o_ref[...] = jnp.dot(a_ref[...], b_ref[...],
                         preferred_element_type=jnp.float32)


def _mm(wt, xh):
    ho, hi = wt.shape
    bn = 2048
    return pl.pallas_call(
        _mm_body,
        grid=(NPAD // bn,),
        in_specs=[
            pl.BlockSpec((ho, hi), lambda j: (0, 0)),
            pl.BlockSpec((hi, bn), lambda j: (0, j)),
        ],
        out_specs=pl.BlockSpec((ho, bn), lambda j: (0, j)),
        out_shape=jax.ShapeDtypeStruct((ho, NPAD), jnp.float32),
    )(wt, xh)


_CH = 4096


@functools.lru_cache(maxsize=None)
def _make_prop(h_out, r_res, pool):
    sweeps = h_out // (NW * r_res)
    assert sweeps * r_res * NW == h_out

    scratch = (
        [pltpu.VMEM((NPAD,), jnp.float32) for _ in range(2 * r_res)]
        + [
            pltpu.VMEM((NPAD,), jnp.float32),
            pltpu.VMEM((h_out,), jnp.float32),
            pltpu.VMEM((_CH,), jnp.int32),
            pltpu.VMEM((_CH,), jnp.float32),
            pltpu.VMEM((_CH,), jnp.int32),
            pltpu.VMEM((_CH,), jnp.float32),
            pltpu.SemaphoreType.DMA,
            pltpu.SemaphoreType.DMA,
        ]
    )
    if pool:
        scratch += [
            pltpu.VMEM((NPAD,), jnp.int32),
            pltpu.VMEM((32,), jnp.float32),
        ]
        out_type = jax.ShapeDtypeStruct((h_out, 32), jnp.float32)
    else:
        out_type = jax.ShapeDtypeStruct((h_out, NPAD), jnp.float32)

    def body(*refs):
        if pool:
            (z_hbm, dinv_hbm, norm_hbm, packed_hbm, b_hbm, batch_hbm,
             out_hbm) = refs[:7]
            sc = refs[7:]
        else:
            (z_hbm, dinv_hbm, norm_hbm, packed_hbm, b_hbm,
             out_hbm) = refs[:6]
            sc = refs[6:]
        ins = sc[:r_res]
        outs = sc[r_res:2 * r_res]
        dinv_v = sc[2 * r_res]
        b_v = sc[2 * r_res + 1]
        ebufs = [sc[2 * r_res + 2:2 * r_res + 4],
                 sc[2 * r_res + 4:2 * r_res + 6]]
        sems = [sc[2 * r_res + 6], sc[2 * r_res + 7]]
        if pool:
            batch_v = sc[2 * r_res + 8]
            acc_v = sc[2 * r_res + 9]

        w = _worker_id()
        pltpu.sync_copy(dinv_hbm, dinv_v)
        pltpu.sync_copy(b_hbm, b_v)
        if pool:
            pltpu.sync_copy(batch_hbm, batch_v)

        for s in range(sweeps):
            feats = [(s * r_res + r) * NW + w for r in range(r_res)]
            for r in range(r_res):
                pltpu.sync_copy(z_hbm.at[feats[r]], ins[r])

            @plsc.parallel_loop(0, NPAD // LANES, unroll=4)
            def init_body(i):
                idx = pl.ds(i * LANES, LANES)
                dv = dinv_v[idx]
                d2 = dv * dv
                for r in range(r_res):
                    outs[r][idx] = ins[r][idx] * d2

            nc = EPAD // _CH

            def issue(c, bset):
                pltpu.async_copy(packed_hbm.at[pl.ds(c * _CH, _CH)],
                                 ebufs[bset][0], sems[bset])
                pltpu.async_copy(norm_hbm.at[pl.ds(c * _CH, _CH)],
                                 ebufs[bset][1], sems[bset])

            def drain(bset):
                pltpu.make_async_copy(packed_hbm.at[pl.ds(0, _CH)],
                                      ebufs[bset][0], sems[bset]).wait()
                pltpu.make_async_copy(norm_hbm.at[pl.ds(0, _CH)],
                                      ebufs[bset][1], sems[bset]).wait()

            def compute(bset):
                pv, nv = ebufs[bset]

                @plsc.parallel_loop(0, _CH // LANES, unroll=4)
                def edge_body(g):
                    idx = pl.ds(g * LANES, LANES)
                    pk = pv[idx]
                    sidx = pk & 16383
                    didx = pk >> 14
                    nm = nv[idx]
                    for r in range(r_res):
                        msg = plsc.load_gather(ins[r], [sidx]) * nm
                        plsc.addupdate_scatter(outs[r], [didx], msg)

            issue(0, 0)

            def chunk_body(j, _):
                c0 = j * 2
                issue(c0 + 1, 1)
                drain(0)
                compute(0)
                issue(jnp.minimum(c0 + 2, nc - 1), 0)
                drain(1)
                compute(1)
                return 0

            lax.fori_loop(0, nc // 2, chunk_body, 0)
            drain(0)

            for r in range(r_res):
                fsplat = jnp.zeros((LANES,), jnp.int32) + feats[r]
                bvec = plsc.load_gather(b_v, [fsplat])
                if pool:
                    acc_v[pl.ds(0, LANES)] = jnp.zeros((LANES,), jnp.float32)
                    acc_v[pl.ds(LANES, LANES)] = jnp.zeros((LANES,), jnp.float32)

                    @plsc.parallel_loop(0, NPAD // LANES, unroll=4)
                    def ep_body(i):
                        idx = pl.ds(i * LANES, LANES)
                        vals = jnp.maximum(outs[r][idx] + bvec, 0.0)
                        ib = batch_v[idx]
                        plsc.addupdate_scatter(acc_v, [ib], vals)

                    pltpu.sync_copy(acc_v, out_hbm.at[feats[r]])
                else:
                    @plsc.parallel_loop(0, NPAD // LANES, unroll=4)
                    def ep_body(i):
                        idx = pl.ds(i * LANES, LANES)
                        outs[r][idx] = jnp.maximum(outs[r][idx] + bvec, 0.0)

                    pltpu.sync_copy(outs[r], out_hbm.at[feats[r]])

    return pl.kernel(
        body,
        out_type=out_type,
        mesh=_mesh(),
        scratch_types=scratch,
        compiler_params=_SC_PARAMS,
    )


def _seq_body(seq_ref, batch_ref, wih_ref, whh_ref, bih_ref, bhh_ref,
              fcw_ref, fcb_ref, out_ref):
    batch3 = batch_ref[...][None, :, :]
    seg = lax.broadcasted_iota(jnp.int32, (B, NPAD // 128, 128), 0)
    cnt = jnp.sum(jnp.where(batch3 == seg, 1.0, 0.0), axis=(1, 2))
    cntc = jnp.maximum(cnt, 1.0).reshape(B, 1)

    bias = (bih_ref[...] + bhh_ref[...])[None, :]
    wih = wih_ref[...]
    whh = whh_ref[...]
    fcw = fcw_ref[...]
    fcb = fcb_ref[...][None, :]

    h = jnp.zeros((B, 128), jnp.float32)
    c = jnp.zeros((B, 128), jnp.float32)
    dn = (((1,), (1,)), ((), ()))
    for t in range(T):
        xt = seq_ref[t] / cntc
        g = (lax.dot_general(xt, wih, dn, preferred_element_type=jnp.float32)
             + lax.dot_general(h, whh, dn, preferred_element_type=jnp.float32)
             + bias)
        i_ = jax.nn.sigmoid(g[:, 0:128])
        f_ = jax.nn.sigmoid(g[:, 128:256])
        g_ = jnp.tanh(g[:, 256:384])
        o_ = jax.nn.sigmoid(g[:, 384:512])
        c = f_ * c + i_ * g_
        h = o_ * jnp.tanh(c)
        out_ref[t] = jnp.dot(h, fcw, preferred_element_type=jnp.float32) + fcb


def _seq_tc(seq, batch80, w_ih, w_hh, b_ih, b_hh, fc_w, fc_b):
    return pl.pallas_call(
        _seq_body,
        out_shape=jax.ShapeDtypeStruct((T, B, 64), jnp.float32),
    )(seq, batch80, w_ih, w_hh, b_ih, b_hh, fc_w, fc_b)


def kernel(x, W1, b1, W2, b2, W3, b3, W4, b4, W_ih, W_hh, b_ih, b_hh,
           fc_W, fc_b, edge_index, batch):
    xp = jnp.pad(x, ((0, 0), (0, NPAD - N), (0, 0)))
    xt = jnp.swapaxes(xp, 1, 2)
    src = jnp.pad(edge_index[:, 0, :], ((0, 0), (0, EPAD - E)),
                  constant_values=NPAD - 1)
    dst = jnp.pad(edge_index[:, 1, :], ((0, 0), (0, EPAD - E)),
                  constant_values=NPAD - 1)
    batch_pad = jnp.pad(batch, (0, NPAD - N), constant_values=B)
    batch80 = batch_pad.reshape(NPAD // 128, 128)

    w1t, w2t, w3t, w4t = W1.T, W2.T, W3.T, W4.T

    deg_k = _make_deg()
    norm_k = _make_norm()
    prop1 = _make_prop(512, 4, False)
    prop2 = _make_prop(256, 4, False)
    prop3 = _make_prop(128, 4, False)
    prop4 = _make_prop(64, 2, True)

    def step(_, ts):
        xt_t, src_t, dst_t = ts
        part = deg_k(dst_t)
        dinv = _dinv_tc(part).reshape(NPAD)
        norm, packed = norm_k(src_t, dst_t, dinv)
        z1 = _mm(w1t, xt_t)
        h1 = prop1(z1, dinv, norm, packed, b1)
        z2 = _mm(w2t, h1)
        h2 = prop2(z2, dinv, norm, packed, b2)
        z3 = _mm(w3t, h2)
        h3 = prop3(z3, dinv, norm, packed, b3)
        z4 = _mm(w4t, h3)
        pool = prop4(z4, dinv, norm, packed, b4, batch_pad)
        return 0, pool

    _, pools = lax.scan(step, 0, (xt, src, dst))
    seq_sums = jnp.transpose(pools, (0, 2, 1))[:, :B, :]

    out8 = _seq_tc(seq_sums, batch80, W_ih, W_hh, b_ih, b_hh, fc_W, fc_b)
    return jnp.swapaxes(out8, 0, 1)
